# baseline (speedup 1.0000x reference)
import functools

import jax
import jax.numpy as jnp
from jax.experimental import pallas as pl
from jax.experimental.pallas import tpu as pltpu

_F = jnp.float32
_BF = jnp.bfloat16


def _bf(x):
    return x.astype(_BF).astype(_F)


def _ln_ax1(x, w_col, b_col, eps=1e-05):
    u = x.mean(axis=1, keepdims=True)
    xc = x - u
    s = (xc * xc).mean(axis=1, keepdims=True)
    inv = 1.0 / jnp.sqrt(s + eps)
    return w_col[None, :, :] * (xc * inv) + b_col[None, :, :]


def _fused_kernel(corr0_ref, corr1_ref, nei_ref, hs_ref, hsT_ref, avT_ref,
                  relW_ref, rel_b_ref, rel_lw_ref, rel_lb_ref,
                  gW_ref, gWT_ref, gate_b_ref, gate_lw_ref, gate_lb_ref,
                  warWC_ref, warW3_ref, war_b_ref,
                  wgtWT_ref, wgt_b_ref, wgt_lw_ref, wgt_lb_ref,
                  out_ref, *, block_i, n, d):
    i = pl.program_id(0)
    hsT = hsT_ref[...]
    hsT_b16 = hsT.astype(_BF)
    av0 = avT_ref[0, :]
    av1 = avT_ref[1, :]
    c0 = corr0_ref[...]
    c1 = corr1_ref[...]

    av2 = av0 * av0 + av1 * av1
    dot = c0 * av0[None, :] + c1 * av1[None, :]
    cc2 = c0 * c0 + c1 * c1
    denom = jnp.sqrt(av2[None, :] * cc2) + 1e-10
    angle = dot / denom
    angle = jnp.where(av2[None, :] == 0.0, -1.0, angle)

    c3 = jnp.concatenate([c0.astype(_BF)[:, None, :],
                          c1.astype(_BF)[:, None, :],
                          angle.astype(_BF)[:, None, :]], axis=1)
    relW_b16 = relW_ref[...].astype(_BF)
    r_pre = jax.lax.dot_general(
        jnp.broadcast_to(relW_b16, (block_i, d, 3)), c3,
        (((2,), (1,)), ((0,), (0,))),
        preferred_element_type=_F) + rel_b_ref[...][None, :, :]
    r = jax.nn.relu(_ln_ax1(r_pre, rel_lw_ref[...], rel_lb_ref[...]))
    r_b16 = r.astype(_BF)

    hs_blk = hs_ref[pl.ds(i * block_i, block_i), :]
    hs_blk_b16 = hs_blk.astype(_BF)

    g1x = jnp.concatenate([gW_ref[:, 0:d].astype(_BF),
                           warW3_ref[0:1, :].astype(_BF)], axis=0)
    rGx = jax.lax.dot_general(
        jnp.broadcast_to(g1x, (block_i, d + 1, d)), r_b16,
        (((2,), (1,)), ((0,), (0,))),
        preferred_element_type=_F)
    rG = rGx[:, 0:d, :]
    tw1 = rGx[:, d, :]
    gIx = jnp.concatenate([gWT_ref[d:2 * d, :].astype(_BF),
                           warWC_ref[:, 1:2].astype(_BF)], axis=1)
    hgIx = jnp.dot(hs_blk_b16, gIx, preferred_element_type=_F)
    g3x = jnp.concatenate([gW_ref[:, 2 * d:3 * d].astype(_BF),
                           warW3_ref[2:3, :].astype(_BF)], axis=0)
    hgJx = jnp.dot(g3x, hsT_b16, preferred_element_type=_F)
    hgJb = hgJx[0:d, :] + gate_b_ref[...]
    gate_pre = rG + hgIx[:, 0:d][:, :, None] + hgJb[None, :, :]
    gu = gate_pre.mean(axis=1, keepdims=True)
    gs = (gate_pre * gate_pre).mean(axis=1, keepdims=True) - gu * gu
    ginv = jax.lax.rsqrt(gs + 1e-05)
    nGate = jax.nn.sigmoid(gate_lw_ref[...][None, :, :]
                           * ((gate_pre - gu) * ginv)
                           + gate_lb_ref[...][None, :, :])

    hwI = hgIx[:, d:d + 1]
    hwJ = hgJx[d:d + 1, :]
    tt = jax.nn.relu((tw1 + hwI + hwJ) + war_b_ref[0, 0])

    mask = nei_ref[...] > 0
    pos_t = jnp.where(mask, tt, 0.0)
    logits = jnp.where(pos_t == 0.0, -10000.0, pos_t)
    m = logits.max(axis=1, keepdims=True)
    e = jnp.exp(logits - m)
    pos = e / e.sum(axis=1, keepdims=True)
    coef = jnp.where(mask, pos, 0.0)

    b_mat = hsT[None, :, :] * nGate
    agg = jax.lax.dot_general(
        b_mat, coef, (((2,), (1,)), ((0,), (0,))),
        preferred_element_type=_F)

    h = jnp.dot(agg.astype(_BF), wgtWT_ref[...].astype(_BF),
                preferred_element_type=_F) + wgt_b_ref[0, :]
    u = h.mean(-1, keepdims=True)
    s = ((h - u) ** 2).mean(-1, keepdims=True)
    h = wgt_lw_ref[0, :] * (h - u) / jnp.sqrt(s + 1e-05) + wgt_lb_ref[0, :]
    out_ref[...] = hs_blk + jax.nn.relu(h)


def kernel(corr_index, nei_index, nei_num, hidden_state, agent_v, rel_W, rel_b, rel_lw, rel_lb, gate_W, gate_b, gate_lw, gate_lb, war_W, war_b, wgt_W, wgt_b, wgt_lw, wgt_lb):
    n, d = hidden_state.shape
    block_i = 64
    grid = (n // block_i,)

    corr0 = corr_index[:, :, 0]
    corr1 = corr_index[:, :, 1]
    avT = agent_v.T
    hsT = hidden_state.T
    gWT = gate_W.T
    wgtWT = wgt_W.T
    warW3 = war_W.reshape(3, d)
    warWC = warW3.T
    col = lambda v: v.reshape(-1, 1)
    row = lambda v: v.reshape(1, -1)

    full2 = lambda a, b: pl.BlockSpec((a, b), lambda i: (0, 0))
    out = pl.pallas_call(
        functools.partial(_fused_kernel, block_i=block_i, n=n, d=d),
        grid=grid,
        in_specs=[
            pl.BlockSpec((block_i, n), lambda i: (i, 0)),
            pl.BlockSpec((block_i, n), lambda i: (i, 0)),
            pl.BlockSpec((block_i, n), lambda i: (i, 0)),
            full2(n, d),
            full2(d, n),
            full2(2, n),
            full2(d, 3),
            full2(d, 1), full2(d, 1), full2(d, 1),
            full2(d, 3 * d),
            full2(3 * d, d),
            full2(d, 1), full2(d, 1), full2(d, 1),
            full2(d, 3),
            full2(3, d),
            full2(1, 1),
            full2(d, d),
            full2(1, d), full2(1, d), full2(1, d),
        ],
        out_specs=pl.BlockSpec((block_i, d), lambda i: (i, 0)),
        out_shape=jax.ShapeDtypeStruct((n, d), jnp.float32),
        compiler_params=pltpu.CompilerParams(
            dimension_semantics=("parallel",)),
    )(corr0, corr1, nei_index, hidden_state, hsT, avT,
      rel_W, col(rel_b), col(rel_lw), col(rel_lb),
      gate_W, gWT, col(gate_b), col(gate_lw), col(gate_lb),
      warWC, warW3, war_b.reshape(1, 1),
      wgtWT, row(wgt_b), row(wgt_lw), row(wgt_lb))
    return out


# speedup vs baseline: 1.0441x; 1.0441x over previous
import functools

import jax
import jax.numpy as jnp
from jax.experimental import pallas as pl
from jax.experimental.pallas import tpu as pltpu

_F = jnp.float32
_BF = jnp.bfloat16


def _bf(x):
    return x.astype(_BF).astype(_F)


def _ln_ax1(x, w_col, b_col, eps=1e-05):
    u = x.mean(axis=1, keepdims=True)
    xc = x - u
    s = (xc * xc).mean(axis=1, keepdims=True)
    inv = 1.0 / jnp.sqrt(s + eps)
    return w_col[None, :, :] * (xc * inv) + b_col[None, :, :]


def _fused_kernel(corr0_ref, corr1_ref, nei_ref, hs_ref, hsT_ref, avT_ref,
                  relW_ref, rel_b_ref, rel_lw_ref, rel_lb_ref,
                  gW_ref, gWT_ref, gate_b_ref, gate_lw_ref, gate_lb_ref,
                  warWC_ref, warW3_ref, war_b_ref,
                  wgtWT_ref, wgt_b_ref, wgt_lw_ref, wgt_lb_ref,
                  out_ref, *, block_i, n, d):
    i = pl.program_id(0)
    hsT = hsT_ref[...]
    hsT_b16 = hsT.astype(_BF)
    av0 = avT_ref[0, :]
    av1 = avT_ref[1, :]
    c0 = corr0_ref[...]
    c1 = corr1_ref[...]

    av2 = av0 * av0 + av1 * av1
    dot = c0 * av0[None, :] + c1 * av1[None, :]
    cc2 = c0 * c0 + c1 * c1
    denom = jnp.sqrt(av2[None, :] * cc2) + 1e-10
    angle = dot / denom
    angle = jnp.where(av2[None, :] == 0.0, -1.0, angle)

    c3 = jnp.concatenate([c0.astype(_BF)[:, None, :],
                          c1.astype(_BF)[:, None, :],
                          angle.astype(_BF)[:, None, :]], axis=1)
    relW_b16 = relW_ref[...].astype(_BF)
    r_pre = jax.lax.dot_general(
        jnp.broadcast_to(relW_b16, (block_i, d, 3)), c3,
        (((2,), (1,)), ((0,), (0,))),
        preferred_element_type=_F) + rel_b_ref[...][None, :, :]
    r = jax.nn.relu(_ln_ax1(r_pre, rel_lw_ref[...], rel_lb_ref[...]))
    r_b16 = r.astype(_BF)

    hs_blk = hs_ref[pl.ds(i * block_i, block_i), :]
    hs_blk_b16 = hs_blk.astype(_BF)

    g1x = jnp.concatenate([gW_ref[:, 0:d].astype(_BF),
                           warW3_ref[0:1, :].astype(_BF)], axis=0)
    rGx = jax.lax.dot_general(
        jnp.broadcast_to(g1x, (block_i, d + 1, d)), r_b16,
        (((2,), (1,)), ((0,), (0,))),
        preferred_element_type=_F)
    rG = rGx[:, 0:d, :]
    tw1 = rGx[:, d, :]
    gIx = jnp.concatenate([gWT_ref[d:2 * d, :].astype(_BF),
                           warWC_ref[:, 1:2].astype(_BF)], axis=1)
    hgIx = jnp.dot(hs_blk_b16, gIx, preferred_element_type=_F)
    g3x = jnp.concatenate([gW_ref[:, 2 * d:3 * d].astype(_BF),
                           warW3_ref[2:3, :].astype(_BF)], axis=0)
    hgJx = jnp.dot(g3x, hsT_b16, preferred_element_type=_F)
    hgJb = hgJx[0:d, :] + gate_b_ref[...]
    gate_pre = rG + hgIx[:, 0:d][:, :, None] + hgJb[None, :, :]
    gu = gate_pre.mean(axis=1, keepdims=True)
    gs = (gate_pre * gate_pre).mean(axis=1, keepdims=True) - gu * gu
    ginv = jax.lax.rsqrt(gs + 1e-05)
    nGate = jax.nn.sigmoid(gate_lw_ref[...][None, :, :]
                           * ((gate_pre - gu) * ginv)
                           + gate_lb_ref[...][None, :, :])

    hwI = hgIx[:, d:d + 1]
    hwJ = hgJx[d:d + 1, :]
    tt = jax.nn.relu((tw1 + hwI + hwJ) + war_b_ref[0, 0])

    mask = nei_ref[...] > 0
    pos_t = jnp.where(mask, tt, 0.0)
    logits = jnp.where(pos_t == 0.0, -10000.0, pos_t)
    m = logits.max(axis=1, keepdims=True)
    e = jnp.exp(logits - m)
    pos = e / e.sum(axis=1, keepdims=True)
    coef = jnp.where(mask, pos, 0.0)

    p = (hsT[None, :, :] * nGate) * coef[:, None, :]
    agg = jnp.sum(p, axis=2)

    h = jnp.dot(agg.astype(_BF), wgtWT_ref[...].astype(_BF),
                preferred_element_type=_F) + wgt_b_ref[0, :]
    u = h.mean(-1, keepdims=True)
    s = ((h - u) ** 2).mean(-1, keepdims=True)
    h = wgt_lw_ref[0, :] * (h - u) / jnp.sqrt(s + 1e-05) + wgt_lb_ref[0, :]
    out_ref[...] = hs_blk + jax.nn.relu(h)


def kernel(corr_index, nei_index, nei_num, hidden_state, agent_v, rel_W, rel_b, rel_lw, rel_lb, gate_W, gate_b, gate_lw, gate_lb, war_W, war_b, wgt_W, wgt_b, wgt_lw, wgt_lb):
    n, d = hidden_state.shape
    block_i = 64
    grid = (n // block_i,)

    corr0 = corr_index[:, :, 0]
    corr1 = corr_index[:, :, 1]
    avT = agent_v.T
    hsT = hidden_state.T
    gWT = gate_W.T
    wgtWT = wgt_W.T
    warW3 = war_W.reshape(3, d)
    warWC = warW3.T
    col = lambda v: v.reshape(-1, 1)
    row = lambda v: v.reshape(1, -1)

    full2 = lambda a, b: pl.BlockSpec((a, b), lambda i: (0, 0))
    out = pl.pallas_call(
        functools.partial(_fused_kernel, block_i=block_i, n=n, d=d),
        grid=grid,
        in_specs=[
            pl.BlockSpec((block_i, n), lambda i: (i, 0)),
            pl.BlockSpec((block_i, n), lambda i: (i, 0)),
            pl.BlockSpec((block_i, n), lambda i: (i, 0)),
            full2(n, d),
            full2(d, n),
            full2(2, n),
            full2(d, 3),
            full2(d, 1), full2(d, 1), full2(d, 1),
            full2(d, 3 * d),
            full2(3 * d, d),
            full2(d, 1), full2(d, 1), full2(d, 1),
            full2(d, 3),
            full2(3, d),
            full2(1, 1),
            full2(d, d),
            full2(1, d), full2(1, d), full2(1, d),
        ],
        out_specs=pl.BlockSpec((block_i, d), lambda i: (i, 0)),
        out_shape=jax.ShapeDtypeStruct((n, d), jnp.float32),
        compiler_params=pltpu.CompilerParams(
            dimension_semantics=("parallel",)),
    )(corr0, corr1, nei_index, hidden_state, hsT, avT,
      rel_W, col(rel_b), col(rel_lw), col(rel_lb),
      gate_W, gWT, col(gate_b), col(gate_lw), col(gate_lb),
      warWC, warW3, war_b.reshape(1, 1),
      wgtWT, row(wgt_b), row(wgt_lw), row(wgt_lb))
    return out
